# Initial kernel scaffold; baseline (speedup 1.0000x reference)
import jax
import jax.numpy as jnp
from jax.experimental import pallas as pl


def kernel(x, table):
    raise NotImplementedError("write your pallas kernel here")


# speedup vs baseline: 1.8377x; 1.8377x over previous
import functools

import jax
import jax.numpy as jnp
from jax import lax
from jax.experimental import pallas as pl
from jax.experimental.pallas import tpu as pltpu
from jax.experimental.pallas import tpu_sc as plsc

D_MODEL = 64
NW = 32
CH = 128
NBUF = 2


def _embed_body(idx_hbm, table_hbm, out_hbm, idx_v, rows_v, sem0, sem1):
    wid = lax.axis_index("s") * 2 + lax.axis_index("c")
    steps = idx_v.shape[0]
    out_base = wid * (steps * CH)

    pltpu.sync_copy(idx_hbm.at[wid], idx_v)

    sems = (sem0, sem1)

    for b in range(NBUF):
        pltpu.async_copy(table_hbm.at[idx_v.at[b]], rows_v.at[b], sems[b])

    def outer(o, carry):
        for b in range(NBUF):
            step = o * NBUF + b
            pltpu.make_async_copy(
                table_hbm.at[idx_v.at[0]], rows_v.at[b], sems[b]
            ).wait()
            pltpu.sync_copy(
                rows_v.at[b], out_hbm.at[pl.ds(out_base + step * CH, CH)]
            )
            nxt = step + NBUF

            @pl.when(nxt < steps)
            def _():
                pltpu.async_copy(
                    table_hbm.at[idx_v.at[nxt]], rows_v.at[b], sems[b]
                )

        return carry

    lax.fori_loop(0, steps // NBUF, outer, 0)


def kernel(x, table):
    B = x.shape[0] * x.shape[1]
    idx = x.reshape(B).astype(jnp.int32)
    b_per_w = B // NW
    steps = b_per_w // CH
    idx3d = idx.reshape(NW, steps, CH)

    mesh = plsc.VectorSubcoreMesh(core_axis_name="c", subcore_axis_name="s")
    run = functools.partial(
        pl.kernel,
        mesh=mesh,
        compiler_params=pltpu.CompilerParams(use_tc_tiling_on_sc=False),
        out_type=jax.ShapeDtypeStruct((B, D_MODEL), jnp.float32),
        scratch_types=[
            pltpu.VMEM((steps, CH), jnp.int32),
            pltpu.VMEM((NBUF, CH, D_MODEL), jnp.float32),
            pltpu.SemaphoreType.DMA,
            pltpu.SemaphoreType.DMA,
        ],
    )(_embed_body)

    out = run(idx3d, table)
    return out.reshape(x.shape[0], x.shape[1], D_MODEL)


# speedup vs baseline: 1.8419x; 1.0023x over previous
import functools

import jax
import jax.numpy as jnp
from jax import lax
from jax.experimental import pallas as pl
from jax.experimental.pallas import tpu as pltpu
from jax.experimental.pallas import tpu_sc as plsc

D_MODEL = 64
NW = 32
NBUF = 4


def _embed_body(x_hbm, table_hbm, out_hbm, idx_v, rows_v, *sems):
    wid = lax.axis_index("s") * 2 + lax.axis_index("c")
    rows_per_w = idx_v.shape[0]
    row_base = wid * rows_per_w

    pltpu.sync_copy(x_hbm.at[pl.ds(row_base, rows_per_w)], idx_v)

    def gather(step, buf):
        return pltpu.async_copy(
            table_hbm.at[idx_v.at[step]], rows_v.at[buf], sems[buf]
        )

    for b in range(NBUF):
        gather(b, b)

    def outer(o, carry):
        for b in range(NBUF):
            step = o * NBUF + b
            pltpu.make_async_copy(
                table_hbm.at[idx_v.at[0]], rows_v.at[b], sems[b]
            ).wait()
            pltpu.sync_copy(rows_v.at[b], out_hbm.at[row_base + step])
            nxt = step + NBUF

            @pl.when(nxt < rows_per_w)
            def _():
                gather(nxt, b)

        return carry

    lax.fori_loop(0, rows_per_w // NBUF, outer, 0)


def kernel(x, table):
    n_rows, n_cols = x.shape
    xi = x.astype(jnp.int32)
    rows_per_w = n_rows // NW

    mesh = plsc.VectorSubcoreMesh(core_axis_name="c", subcore_axis_name="s")
    run = functools.partial(
        pl.kernel,
        mesh=mesh,
        compiler_params=pltpu.CompilerParams(use_tc_tiling_on_sc=False),
        out_type=jax.ShapeDtypeStruct((n_rows, n_cols, D_MODEL), jnp.float32),
        scratch_types=[
            pltpu.VMEM((rows_per_w, n_cols), jnp.int32),
            pltpu.VMEM((NBUF, n_cols, D_MODEL), jnp.float32),
        ]
        + [pltpu.SemaphoreType.DMA] * NBUF,
    )(_embed_body)

    return run(xi, table)
